# Initial kernel scaffold; baseline (speedup 1.0000x reference)
import jax
import jax.numpy as jnp
from jax.experimental import pallas as pl


def kernel(x, edge_index, ln_weight, ln_bias, W, gcn_bias):
    raise NotImplementedError("write your pallas kernel here")


# speedup vs baseline: 11.1173x; 11.1173x over previous
import functools

import jax
import jax.numpy as jnp
from jax import lax
from jax.experimental import pallas as pl
from jax.experimental.pallas import tpu as pltpu
from jax.experimental.pallas import tpu_sc as plsc

N = 10000
H = 256
HH = 128
E = 160000
CHUNK = 128
E_PAD = 163840
N_CHUNKS = E_PAD // CHUNK
NPAD = 10112
STRIPE = NPAD // 16
DEG_W = 16
DEG_CHUNKS = N_CHUNKS // 32
PER_TILE = N_CHUNKS // 16

_sc_mesh = plsc.VectorSubcoreMesh(core_axis_name="c", subcore_axis_name="s")


@functools.partial(
    pl.kernel,
    out_type=jax.ShapeDtypeStruct((2, NPAD, DEG_W), jnp.float32),
    mesh=_sc_mesh,
    scratch_types=[
        pltpu.VMEM((8, CHUNK), jnp.int32),
        pltpu.VMEM((CHUNK, DEG_W), jnp.float32),
        pltpu.VMEM_SHARED((NPAD, DEG_W), jnp.float32),
        [pltpu.SemaphoreType.DMA] * 8,
        [pltpu.SemaphoreType.DMA] * 4,
    ],
)
def _deg_kernel(dst_hbm, zer_hbm, out_hbm, idx8, ones_v, acc_sh, isem, asem):
    c = lax.axis_index("c")
    s = lax.axis_index("s")
    wid = s * 2 + c

    def fill_ones(i, _):
        ones_v[i, :] = jnp.ones((16,), jnp.float32)
        return 0

    lax.fori_loop(0, CHUNK, fill_ones, 0)
    base = s * STRIPE
    pltpu.sync_copy(zer_hbm, acc_sh.at[pl.ds(base, STRIPE)])
    plsc.subcore_barrier()

    for j in range(4):
        pltpu.async_copy(dst_hbm.at[wid, j], idx8.at[j], isem[j])

    def outer(i, _):
        for b in range(8):
            k = i * 8 + b

            @pl.when(k >= 4)
            def _wait_add():
                pltpu.make_async_copy(
                    ones_v, acc_sh.at[idx8.at[(b + 4) % 8]], asem[b % 4]).wait()

            @pl.when(k + 4 < DEG_CHUNKS)
            def _load_idx():
                pltpu.async_copy(dst_hbm.at[wid, k + 4],
                                 idx8.at[(b + 4) % 8], isem[(b + 4) % 8])

            pltpu.make_async_copy(dst_hbm.at[wid, 0], idx8.at[b], isem[b]).wait()
            pltpu.async_copy(ones_v, acc_sh.at[idx8.at[b]], asem[b % 4], add=True)
        return 0

    lax.fori_loop(0, DEG_CHUNKS // 8, outer, 0)
    for b in range(4):
        pltpu.make_async_copy(
            ones_v, acc_sh.at[idx8.at[4 + b]], asem[b]).wait()
    plsc.subcore_barrier()
    pltpu.sync_copy(acc_sh.at[pl.ds(base, STRIPE)],
                    out_hbm.at[c, pl.ds(base, STRIPE)])


@functools.partial(
    pl.kernel,
    out_type=jax.ShapeDtypeStruct((2, NPAD, HH), jnp.float32),
    mesh=_sc_mesh,
    scratch_types=[
        pltpu.VMEM((4, 3, CHUNK), jnp.int32),
        [pltpu.VMEM((CHUNK, HH), jnp.float32)] * 2,
        pltpu.VMEM_SHARED((NPAD, HH), jnp.float32),
        [pltpu.SemaphoreType.DMA] * 4,
        [pltpu.SemaphoreType.DMA] * 2,
        [pltpu.SemaphoreType.DMA] * 2,
    ],
)
def _edges_kernel(h_hbm, pk_hbm, zer_hbm, out_hbm,
                  idx4, rows, acc, isem, gsem, ssem):
    c = lax.axis_index("c")
    s = lax.axis_index("s")
    base = s * STRIPE
    pltpu.sync_copy(zer_hbm, acc.at[pl.ds(base, STRIPE)])
    plsc.subcore_barrier()

    for j in range(3):
        pltpu.async_copy(pk_hbm.at[s, j], idx4.at[j], isem[j])
    pltpu.make_async_copy(pk_hbm.at[s, 0], idx4.at[0], isem[0]).wait()
    pltpu.async_copy(h_hbm.at[idx4.at[0, c]], rows[0], gsem[0])

    def outer(i, _):
        for b in range(4):
            k = i * 4 + b
            rb = b % 2

            @pl.when(k >= 1)
            def _wait_scatter_prev():
                pltpu.make_async_copy(
                    rows[1 - rb], acc.at[idx4.at[(b + 3) % 4, 2]],
                    ssem[1 - rb]).wait()

            @pl.when(k + 1 < PER_TILE)
            def _next_gather():
                pltpu.make_async_copy(
                    pk_hbm.at[s, 0], idx4.at[(b + 1) % 4],
                    isem[(b + 1) % 4]).wait()
                pltpu.async_copy(h_hbm.at[idx4.at[(b + 1) % 4, c]],
                                 rows[1 - rb], gsem[1 - rb])

            @pl.when(k + 3 < PER_TILE)
            def _load_idx():
                pltpu.async_copy(pk_hbm.at[s, k + 3],
                                 idx4.at[(b + 3) % 4], isem[(b + 3) % 4])

            pltpu.make_async_copy(h_hbm.at[idx4.at[b, c]], rows[rb],
                                  gsem[rb]).wait()
            pltpu.async_copy(rows[rb], acc.at[idx4.at[b, 2]], ssem[rb],
                             add=True)
        return 0

    lax.fori_loop(0, PER_TILE // 4, outer, 0)
    pltpu.make_async_copy(rows[1], acc.at[idx4.at[3, 2]], ssem[1]).wait()
    plsc.subcore_barrier()
    pltpu.sync_copy(acc.at[pl.ds(base, STRIPE)],
                    out_hbm.at[c, pl.ds(base, STRIPE)])


_R = 1000


def _dense1_body(x_ref, w_ref, lnw_ref, lnb_ref, deg_ref, h2_ref):
    xb = x_ref[...]
    mu = jnp.mean(xb, axis=-1, keepdims=True)
    var = jnp.mean((xb - mu) ** 2, axis=-1, keepdims=True)
    gate = (xb - mu) * lax.rsqrt(var + 1e-5) * lnw_ref[...] + lnb_ref[...]
    h = lax.dot_general(gate, w_ref[...], (((1,), (1,)), ((), ())),
                        preferred_element_type=jnp.float32)
    hp = h * lax.rsqrt(deg_ref[...])
    h2_ref[...] = jnp.stack([hp[:, :HH], hp[:, HH:]], axis=0)


def _dense1(x, W, lnw, lnb, deg):
    return pl.pallas_call(
        _dense1_body,
        grid=(N // _R,),
        in_specs=[
            pl.BlockSpec((_R, H), lambda i: (i, 0)),
            pl.BlockSpec((H, H), lambda i: (0, 0)),
            pl.BlockSpec((1, H), lambda i: (0, 0)),
            pl.BlockSpec((1, H), lambda i: (0, 0)),
            pl.BlockSpec((_R, 1), lambda i: (i, 0)),
        ],
        out_specs=pl.BlockSpec((2, _R, HH), lambda i: (0, i, 0)),
        out_shape=jax.ShapeDtypeStruct((2, N, HH), jnp.float32),
    )(x, W, lnw, lnb, deg)


def _dense2_body(s2_ref, h2_ref, deg_ref, x_ref, b_ref, o_ref):
    tot = s2_ref[...] + h2_ref[...]
    t = jnp.concatenate([tot[0], tot[1]], axis=-1)
    out = t * lax.rsqrt(deg_ref[...]) + b_ref[...]
    o_ref[...] = jnp.tanh(out) * x_ref[...]


def _dense2(s2, h2, deg, x, bias):
    return pl.pallas_call(
        _dense2_body,
        grid=(N // _R,),
        in_specs=[
            pl.BlockSpec((2, _R, HH), lambda i: (0, i, 0)),
            pl.BlockSpec((2, _R, HH), lambda i: (0, i, 0)),
            pl.BlockSpec((_R, 1), lambda i: (i, 0)),
            pl.BlockSpec((_R, H), lambda i: (i, 0)),
            pl.BlockSpec((1, H), lambda i: (0, 0)),
        ],
        out_specs=pl.BlockSpec((_R, H), lambda i: (i, 0)),
        out_shape=jax.ShapeDtypeStruct((N, H), jnp.float32),
    )(s2, h2, deg, x, bias)


def kernel(x, edge_index, ln_weight, ln_bias, W, gcn_bias):
    src = edge_index[0].astype(jnp.int32)
    dst = edge_index[1].astype(jnp.int32)
    pad = E_PAD - E
    srcp = jnp.concatenate([src, jnp.zeros((pad,), jnp.int32)])
    dstp = jnp.concatenate([dst, jnp.full((pad,), N, jnp.int32)])
    pk = jnp.stack([srcp.reshape(16, PER_TILE, CHUNK),
                    srcp.reshape(16, PER_TILE, CHUNK) + N,
                    dstp.reshape(16, PER_TILE, CHUNK)], axis=2)
    dst_deg = dstp.reshape(32, DEG_CHUNKS, CHUNK)
    zer_deg = jnp.zeros((STRIPE, DEG_W), jnp.float32)
    zer_row = jnp.zeros((STRIPE, HH), jnp.float32)

    deg_parts = _deg_kernel(dst_deg, zer_deg)
    deg = (1.0 + deg_parts[0, :N, 0] + deg_parts[1, :N, 0]).reshape(N, 1)

    h2 = _dense1(x, W, ln_weight.reshape(1, H), ln_bias.reshape(1, H), deg)
    h_flat = h2.reshape(2 * N, HH)

    s2 = _edges_kernel(h_flat, pk, zer_row)

    return _dense2(s2, h2, deg, x, gcn_bias.reshape(1, H))


# speedup vs baseline: 11.2227x; 1.0095x over previous
import functools

import jax
import jax.numpy as jnp
from jax import lax
from jax.experimental import pallas as pl
from jax.experimental.pallas import tpu as pltpu
from jax.experimental.pallas import tpu_sc as plsc

N = 10000
H = 256
HH = 128
E = 160000
CHUNK = 128
E_PAD = 163840
N_CHUNKS = E_PAD // CHUNK
NPAD = 10112
STRIPE = NPAD // 16
DEG_W = 16
DEG_CHUNKS = N_CHUNKS // 32
PER_TILE = N_CHUNKS // 16
Q = 32
QUARTERS = PER_TILE * 4

_sc_mesh = plsc.VectorSubcoreMesh(core_axis_name="c", subcore_axis_name="s")


@functools.partial(
    pl.kernel,
    out_type=jax.ShapeDtypeStruct((2, NPAD, DEG_W), jnp.float32),
    mesh=_sc_mesh,
    scratch_types=[
        pltpu.VMEM((8, CHUNK), jnp.int32),
        pltpu.VMEM((CHUNK, DEG_W), jnp.float32),
        pltpu.VMEM_SHARED((NPAD, DEG_W), jnp.float32),
        [pltpu.SemaphoreType.DMA] * 8,
        [pltpu.SemaphoreType.DMA] * 4,
    ],
)
def _deg_kernel(dst_hbm, zer_hbm, out_hbm, idx8, ones_v, acc_sh, isem, asem):
    c = lax.axis_index("c")
    s = lax.axis_index("s")
    wid = s * 2 + c

    def fill_ones(i, _):
        ones_v[i, :] = jnp.ones((16,), jnp.float32)
        return 0

    lax.fori_loop(0, CHUNK, fill_ones, 0)
    base = s * STRIPE
    pltpu.sync_copy(zer_hbm, acc_sh.at[pl.ds(base, STRIPE)])
    plsc.subcore_barrier()

    for j in range(4):
        pltpu.async_copy(dst_hbm.at[wid, j], idx8.at[j], isem[j])

    def outer(i, _):
        for b in range(8):
            k = i * 8 + b

            @pl.when(k >= 4)
            def _wait_add():
                pltpu.make_async_copy(
                    ones_v, acc_sh.at[idx8.at[(b + 4) % 8]], asem[b % 4]).wait()

            @pl.when(k + 4 < DEG_CHUNKS)
            def _load_idx():
                pltpu.async_copy(dst_hbm.at[wid, k + 4],
                                 idx8.at[(b + 4) % 8], isem[(b + 4) % 8])

            pltpu.make_async_copy(dst_hbm.at[wid, 0], idx8.at[b], isem[b]).wait()
            pltpu.async_copy(ones_v, acc_sh.at[idx8.at[b]], asem[b % 4], add=True)
        return 0

    lax.fori_loop(0, DEG_CHUNKS // 8, outer, 0)
    for b in range(4):
        pltpu.make_async_copy(
            ones_v, acc_sh.at[idx8.at[4 + b]], asem[b]).wait()
    plsc.subcore_barrier()
    pltpu.sync_copy(acc_sh.at[pl.ds(base, STRIPE)],
                    out_hbm.at[c, pl.ds(base, STRIPE)])


@functools.partial(
    pl.kernel,
    out_type=jax.ShapeDtypeStruct((2, NPAD, HH), jnp.float32),
    mesh=_sc_mesh,
    scratch_types=[
        pltpu.VMEM((8, 3, CHUNK), jnp.int32),
        [pltpu.VMEM((Q, HH), jnp.float32)] * 8,
        pltpu.VMEM_SHARED((NPAD, HH), jnp.float32),
        [pltpu.SemaphoreType.DMA] * 8,
        [pltpu.SemaphoreType.DMA] * 8,
        [pltpu.SemaphoreType.DMA] * 8,
    ],
)
def _edges_kernel(h_hbm, pk_hbm, zer_hbm, out_hbm,
                  idx8, rows, acc, isem, gsem, ssem):
    c = lax.axis_index("c")
    s = lax.axis_index("s")
    base = s * STRIPE
    pltpu.sync_copy(zer_hbm, acc.at[pl.ds(base, STRIPE)])
    plsc.subcore_barrier()

    def q_src(bf, par):
        return idx8.at[bf, c, pl.ds(par * Q, Q)]

    def q_dst(bf, par):
        return acc.at[idx8.at[bf, 2, pl.ds(par * Q, Q)]]

    for j in range(4):
        pltpu.async_copy(pk_hbm.at[s, j], idx8.at[j], isem[j])
    for j in range(5):
        if j % 4 == 0:
            pltpu.make_async_copy(
                pk_hbm.at[s, 0], idx8.at[j // 4], isem[j // 4]).wait()
        pltpu.async_copy(h_hbm.at[q_src(j // 4, j % 4)], rows[j], gsem[j])

    def outer(i, _):
        for b in range(32):
            m = i * 32 + b

            @pl.when(m >= 2)
            def _retire_scatter():
                p = (b + 30) % 32
                pltpu.make_async_copy(
                    rows[(b + 6) % 8], q_dst(p // 4, p % 4),
                    ssem[(b + 6) % 8]).wait()

            @pl.when(m + 5 < QUARTERS)
            def _next_gather():
                q = (b + 5) % 32
                if q % 4 == 0:
                    pltpu.make_async_copy(
                        pk_hbm.at[s, 0], idx8.at[q // 4], isem[q // 4]).wait()
                pltpu.async_copy(h_hbm.at[q_src(q // 4, q % 4)],
                                 rows[(b + 5) % 8], gsem[(b + 5) % 8])

            if b % 4 == 0:
                bf = b // 4

                @pl.when(m // 4 + 4 < PER_TILE)
                def _load_idx():
                    pltpu.async_copy(pk_hbm.at[s, m // 4 + 4],
                                     idx8.at[(bf + 4) % 8],
                                     isem[(bf + 4) % 8])

            pltpu.make_async_copy(h_hbm.at[q_src(b // 4, b % 4)],
                                  rows[b % 8], gsem[b % 8]).wait()
            pltpu.async_copy(rows[b % 8], q_dst(b // 4, b % 4),
                             ssem[b % 8], add=True)
        return 0

    lax.fori_loop(0, QUARTERS // 32, outer, 0)
    for m in (QUARTERS - 2, QUARTERS - 1):
        b = m % 32
        pltpu.make_async_copy(rows[b % 8], q_dst(b // 4, b % 4),
                              ssem[b % 8]).wait()
    plsc.subcore_barrier()
    pltpu.sync_copy(acc.at[pl.ds(base, STRIPE)],
                    out_hbm.at[c, pl.ds(base, STRIPE)])


_R = 1000


def _dense1_body(x_ref, w_ref, lnw_ref, lnb_ref, deg_ref, h2_ref):
    xb = x_ref[...]
    mu = jnp.mean(xb, axis=-1, keepdims=True)
    var = jnp.mean((xb - mu) ** 2, axis=-1, keepdims=True)
    gate = (xb - mu) * lax.rsqrt(var + 1e-5) * lnw_ref[...] + lnb_ref[...]
    h = lax.dot_general(gate, w_ref[...], (((1,), (1,)), ((), ())),
                        preferred_element_type=jnp.float32)
    hp = h * lax.rsqrt(deg_ref[...])
    h2_ref[...] = jnp.stack([hp[:, :HH], hp[:, HH:]], axis=0)


def _dense1(x, W, lnw, lnb, deg):
    return pl.pallas_call(
        _dense1_body,
        grid=(N // _R,),
        in_specs=[
            pl.BlockSpec((_R, H), lambda i: (i, 0)),
            pl.BlockSpec((H, H), lambda i: (0, 0)),
            pl.BlockSpec((1, H), lambda i: (0, 0)),
            pl.BlockSpec((1, H), lambda i: (0, 0)),
            pl.BlockSpec((_R, 1), lambda i: (i, 0)),
        ],
        out_specs=pl.BlockSpec((2, _R, HH), lambda i: (0, i, 0)),
        out_shape=jax.ShapeDtypeStruct((2, N, HH), jnp.float32),
    )(x, W, lnw, lnb, deg)


def _dense2_body(s2_ref, h2_ref, deg_ref, x_ref, b_ref, o_ref):
    tot = s2_ref[...] + h2_ref[...]
    t = jnp.concatenate([tot[0], tot[1]], axis=-1)
    out = t * lax.rsqrt(deg_ref[...]) + b_ref[...]
    o_ref[...] = jnp.tanh(out) * x_ref[...]


def _dense2(s2, h2, deg, x, bias):
    return pl.pallas_call(
        _dense2_body,
        grid=(N // _R,),
        in_specs=[
            pl.BlockSpec((2, _R, HH), lambda i: (0, i, 0)),
            pl.BlockSpec((2, _R, HH), lambda i: (0, i, 0)),
            pl.BlockSpec((_R, 1), lambda i: (i, 0)),
            pl.BlockSpec((_R, H), lambda i: (i, 0)),
            pl.BlockSpec((1, H), lambda i: (0, 0)),
        ],
        out_specs=pl.BlockSpec((_R, H), lambda i: (i, 0)),
        out_shape=jax.ShapeDtypeStruct((N, H), jnp.float32),
    )(s2, h2, deg, x, bias)


def kernel(x, edge_index, ln_weight, ln_bias, W, gcn_bias):
    src = edge_index[0].astype(jnp.int32)
    dst = edge_index[1].astype(jnp.int32)
    pad = E_PAD - E
    srcp = jnp.concatenate([src, jnp.zeros((pad,), jnp.int32)])
    dstp = jnp.concatenate([dst, jnp.full((pad,), N, jnp.int32)])
    pk = jnp.stack([srcp.reshape(16, PER_TILE, CHUNK),
                    srcp.reshape(16, PER_TILE, CHUNK) + N,
                    dstp.reshape(16, PER_TILE, CHUNK)], axis=2)
    dst_deg = dstp.reshape(32, DEG_CHUNKS, CHUNK)
    zer_deg = jnp.zeros((STRIPE, DEG_W), jnp.float32)
    zer_row = jnp.zeros((STRIPE, HH), jnp.float32)

    deg_parts = _deg_kernel(dst_deg, zer_deg)
    deg = (1.0 + deg_parts[0, :N, 0] + deg_parts[1, :N, 0]).reshape(N, 1)

    h2 = _dense1(x, W, ln_weight.reshape(1, H), ln_bias.reshape(1, H), deg)
    h_flat = h2.reshape(2 * N, HH)

    s2 = _edges_kernel(h_flat, pk, zer_row)

    return _dense2(s2, h2, deg, x, gcn_bias.reshape(1, H))
